# baseline (speedup 1.0000x reference)
import functools

import jax
import jax.numpy as jnp
from jax import lax
from jax.experimental import pallas as pl
from jax.experimental.pallas import tpu as pltpu
from jax.experimental.pallas import tpu_sc as plsc

N_BOXES = 20000
PRE_NMS_TOP_N = 2000
POST_NMS_TOP_N = 100
NMS_IOU_THRESHOLD = 0.7
SCORE_THRESHOLD = 0.1

NS = 32768
SR, SC_ = 256, 128
KR, KC = 16, 128
K = KR * KC
NEG = -jnp.inf


def _roll(x, s, axis):
  n = x.shape[axis]
  s = s % n
  if s == 0:
    return x
  if axis == 0:
    return jnp.concatenate([x[s:], x[:s]], axis=0)
  return jnp.concatenate([x[:, s:], x[:, :s]], axis=1)


def _bit_mask(shape, dist, rows, cols):
  if dist < cols:
    it = lax.broadcasted_iota(jnp.int32, shape, 1)
    return (it & dist) == 0
  it = lax.broadcasted_iota(jnp.int32, shape, 0)
  return (it & (dist // cols)) == 0


def _partner(x, dist, cols, first_mask):
  if dist < cols:
    fwd = _roll(x, dist, 1)
    bwd = _roll(x, -dist, 1)
  else:
    fwd = _roll(x, dist // cols, 0)
    bwd = _roll(x, -(dist // cols), 0)
  return jnp.where(first_mask, fwd, bwd)


def _stage_dir(arrays, greater, dist, dirmask, rows, cols):
  shape = arrays[0].shape
  first = _bit_mask(shape, dist, rows, cols)
  parts = [_partner(a, dist, cols, first) for a in arrays]
  self_gt = greater(arrays, parts)
  want_self_gt = dirmask == first
  take = want_self_gt != self_gt
  return [jnp.where(take, p, a) for a, p in zip(arrays, parts)]


def _block_sort(arrays, greater, desc, rows, cols):
  n = rows * cols
  shape = arrays[0].shape
  level = 2
  while level <= n:
    dm = _bit_mask(shape, level, rows, cols)
    if not desc:
      dm = ~dm
    dist = level // 2
    while dist >= 1:
      arrays = _stage_dir(arrays, greater, dist, dm, rows, cols)
      dist //= 2
    level *= 2
  return arrays


def _merge_net(arrays, greater, desc, rows, cols):
  shape = arrays[0].shape
  dm = jnp.full(shape, desc, jnp.bool_)
  dist = (rows * cols) // 2
  while dist >= 1:
    arrays = _stage_dir(arrays, greater, dist, dm, rows, cols)
    dist //= 2
  return arrays


def _score_idx_greater(selfs, parts):
  s, i = selfs[0], selfs[1]
  ps, pi = parts[0], parts[1]
  return (s > ps) | ((s == ps) & (i < pi))


def _sort_kernel(scores_ref, ts_ref, ti_ref):
  lidx = (lax.broadcasted_iota(jnp.int32, (KR, KC), 0) * KC
          + lax.broadcasted_iota(jnp.int32, (KR, KC), 1))
  blocks = []
  for i in range(SR // KR):
    s = scores_ref[i * KR:(i + 1) * KR, :]
    s = jnp.where(s > SCORE_THRESHOLD, s, NEG)
    idx = lidx + i * K
    blocks.append(
        _block_sort([s, idx], _score_idx_greater, i % 2 == 0, KR, KC))
  while len(blocks) > 1:
    nxt = []
    for j in range(0, len(blocks), 2):
      a, b = blocks[j], blocks[j + 1]
      gt = _score_idx_greater(a, b)
      m = [jnp.where(gt, x, y) for x, y in zip(a, b)]
      nxt.append(_merge_net(m, _score_idx_greater, len(nxt) % 2 == 0,
                            KR, KC))
    blocks = nxt
  s, idx = blocks[0]
  ts_ref[...] = s
  ti_ref[...] = idx


def _topk_sorted(scores):
  pad = jnp.full((NS - N_BOXES,), 0.0, jnp.float32)
  s2d = jnp.concatenate([scores, pad]).reshape(SR, SC_)
  return pl.pallas_call(
      _sort_kernel,
      out_shape=[
          jax.ShapeDtypeStruct((KR, KC), jnp.float32),
          jax.ShapeDtypeStruct((KR, KC), jnp.int32),
      ],
  )(s2d)


_NG = 4 * K


@functools.cache
def _make_sc_gather():
  info = plsc.get_sparse_core_info()
  nw = info.num_cores * info.num_subcores
  b_per_w = K // nw
  mesh = plsc.VectorSubcoreMesh(core_axis_name="c", subcore_axis_name="s")

  @functools.partial(
      pl.kernel,
      mesh=mesh,
      out_type=jax.ShapeDtypeStruct((_NG,), jnp.float32),
      scratch_types=[
          pltpu.VMEM((b_per_w,), jnp.int32),
          [pltpu.VMEM((b_per_w,), jnp.float32) for _ in range(4)],
          pltpu.SemaphoreType.DMA,
      ],
  )
  def gather(x1h, y1h, x2h, y2h, idx_hbm, out_hbm, idx_v, vals, sem):
    wid = lax.axis_index("s") * info.num_cores + lax.axis_index("c")
    base = wid * b_per_w
    pltpu.sync_copy(idx_hbm.at[pl.ds(base, b_per_w)], idx_v)
    cps = [pltpu.async_copy(src.at[idx_v], buf, sem)
           for src, buf in zip((x1h, y1h, x2h, y2h), vals)]
    for cp in cps:
      cp.wait()
    for c, buf in enumerate(vals):
      pltpu.sync_copy(buf, out_hbm.at[pl.ds(c * K + base, b_per_w)])

  return gather


def _gather_topk_boxes(coords, idx_flat):
  return _make_sc_gather()(*coords, idx_flat)


def _iou_block(x1c, y1c, x2c, y2c, ac, x1r, y1r, x2r, y2r, ar):
  ix1 = jnp.maximum(x1c, x1r)
  iy1 = jnp.maximum(y1c, y1r)
  ix2 = jnp.minimum(x2c, x2r)
  iy2 = jnp.minimum(y2c, y2r)
  iw = jnp.clip(ix2 - ix1, 0.0)
  ih = jnp.clip(iy2 - iy1, 0.0)
  inter = iw * ih
  union = ac + ar - inter
  return inter / jnp.maximum(union, 1e-8)


def _sel_greater(selfs, parts):
  s, p = selfs[0], selfs[1]
  ps, pp = parts[0], parts[1]
  return (s > ps) | ((s == ps) & (p < pp))


def _nms_kernel(ts_ref, g_ref, out_ref, keep_ref):
  ts = ts_ref[...]
  x1 = g_ref[0 * KR:1 * KR]
  y1 = g_ref[1 * KR:2 * KR]
  x2 = g_ref[2 * KR:3 * KR]
  y2 = g_ref[3 * KR:4 * KR]
  area = (x2 - x1) * (y2 - y1)

  pos = (lax.broadcasted_iota(jnp.int32, (KR, KC), 0) * KC
         + lax.broadcasted_iota(jnp.int32, (KR, KC), 1))
  valid = pos < PRE_NMS_TOP_N
  lane = lax.broadcasted_iota(jnp.int32, (1, KC), 1)
  keep_ref[...] = valid.astype(jnp.int32)

  def block_body(carry):
    b, cnt = carry
    x1r = g_ref[pl.ds(0 * KR + b, 1), :]
    y1r = g_ref[pl.ds(1 * KR + b, 1), :]
    x2r = g_ref[pl.ds(2 * KR + b, 1), :]
    y2r = g_ref[pl.ds(3 * KR + b, 1), :]
    arr = (x2r - x1r) * (y2r - y1r)
    x1c, y1c, x2c, y2c, acc = (jnp.transpose(v)
                               for v in (x1r, y1r, x2r, y2r, arr))
    iou_bb = _iou_block(x1c, y1c, x2c, y2c, acc, x1r, y1r, x2r, y2r, arr)

    s_keep0 = (keep_ref[pl.ds(b, 1), :] > 0).astype(jnp.float32)

    sub_i = lax.broadcasted_iota(jnp.int32, (KC, KC), 0)
    lan_i = lax.broadcasted_iota(jnp.int32, (KC, KC), 1)
    adjm = (iou_bb > NMS_IOU_THRESHOLD) & (sub_i < lan_i)

    def jac_body(c):
      kf, _ = c
      kcol = jnp.transpose(kf)
      hit = jnp.any(adjm & (kcol > 0.5), axis=0, keepdims=True)
      knew = jnp.where(hit, 0.0, s_keep0)
      return knew, jnp.sum(jnp.abs(knew - kf)) > 0.0

    skf, _ = lax.while_loop(lambda c: c[1], jac_body, (s_keep0, True))
    s_keep = skf > 0.5

    skc = jnp.transpose(s_keep)
    for rr in range(KR):
      @pl.when(rr > b)
      def _():
        x1t, y1t, x2t, y2t, art = (a[rr:rr + 1]
                                   for a in (x1, y1, x2, y2, area))
        iou_c = _iou_block(x1c, y1c, x2c, y2c, acc, x1t, y1t, x2t, y2t, art)
        hit = jnp.any((iou_c > NMS_IOU_THRESHOLD) & skc, axis=0,
                      keepdims=True)
        krow = keep_ref[rr:rr + 1, :] > 0
        keep_ref[rr:rr + 1, :] = (krow & ~hit).astype(jnp.int32)
    keep_ref[pl.ds(b, 1), :] = s_keep.astype(jnp.int32)
    cnt = cnt + jnp.sum(skf)
    return b + 1, cnt

  def block_cond(carry):
    b, cnt = carry
    return (b < KR) & (cnt < POST_NMS_TOP_N)

  b_fin, _ = lax.while_loop(block_cond, block_body, (0, 0.0))

  keep = keep_ref[...] > 0
  sel = jnp.where(keep & (pos < b_fin * KC), ts, NEG)
  ssel, _, sx1, sy1, sx2, sy2 = _block_sort(
      [sel, pos, x1, y1, x2, y2], _sel_greater, True, KR, KC)

  zero = jnp.zeros((3, KC), jnp.float32)
  out_ref[...] = jnp.concatenate(
      [sx1[:1], sy1[:1], sx2[:1], sy2[:1], ssel[:1], zero], axis=0)


def _nms(ts, g2d):
  return pl.pallas_call(
      _nms_kernel,
      out_shape=jax.ShapeDtypeStruct((8, KC), jnp.float32),
      scratch_shapes=[
          pltpu.VMEM((KR, KC), jnp.int32),
      ],
  )(ts, g2d)


def kernel(boxes, scores):
  ts, ti = _topk_sorted(scores)
  coords = tuple(boxes[:, i] for i in range(4))
  g = _gather_topk_boxes(coords, ti.reshape(K))
  outb = _nms(ts, g.reshape(4 * KR, KC))
  final_boxes = jnp.stack(
      [outb[0, :POST_NMS_TOP_N], outb[1, :POST_NMS_TOP_N],
       outb[2, :POST_NMS_TOP_N], outb[3, :POST_NMS_TOP_N]], axis=1)
  final_scores = outb[4, :POST_NMS_TOP_N]
  return final_boxes, final_scores


# speedup vs baseline: 1.0807x; 1.0807x over previous
import functools

import jax
import jax.numpy as jnp
from jax import lax
from jax.experimental import pallas as pl
from jax.experimental.pallas import tpu as pltpu
from jax.experimental.pallas import tpu_sc as plsc

N_BOXES = 20000
PRE_NMS_TOP_N = 2000
POST_NMS_TOP_N = 100
NMS_IOU_THRESHOLD = 0.7
SCORE_THRESHOLD = 0.1

NS = 32768
SR, SC_ = 256, 128
KR, KC = 16, 128
K = KR * KC
NEG = -jnp.inf


def _roll(x, s, axis):
  n = x.shape[axis]
  s = s % n
  if s == 0:
    return x
  if axis == 0:
    return jnp.concatenate([x[s:], x[:s]], axis=0)
  return jnp.concatenate([x[:, s:], x[:, :s]], axis=1)


def _bit_mask(shape, dist, rows, cols):
  if dist < cols:
    it = lax.broadcasted_iota(jnp.int32, shape, 1)
    return (it & dist) == 0
  it = lax.broadcasted_iota(jnp.int32, shape, 0)
  return (it & (dist // cols)) == 0


def _partner(x, dist, cols, first_mask):
  if dist < cols:
    fwd = _roll(x, dist, 1)
    bwd = _roll(x, -dist, 1)
  else:
    fwd = _roll(x, dist // cols, 0)
    bwd = _roll(x, -(dist // cols), 0)
  return jnp.where(first_mask, fwd, bwd)


def _stage_dir(arrays, greater, dist, dirmask, rows, cols):
  shape = arrays[0].shape
  first = _bit_mask(shape, dist, rows, cols)
  parts = [_partner(a, dist, cols, first) for a in arrays]
  self_gt = greater(arrays, parts)
  want_self_gt = dirmask == first
  take = want_self_gt != self_gt
  return [jnp.where(take, p, a) for a, p in zip(arrays, parts)]


def _block_sort(arrays, greater, desc, rows, cols):
  n = rows * cols
  shape = arrays[0].shape
  level = 2
  while level <= n:
    dm = _bit_mask(shape, level, rows, cols)
    if not desc:
      dm = ~dm
    dist = level // 2
    while dist >= 1:
      arrays = _stage_dir(arrays, greater, dist, dm, rows, cols)
      dist //= 2
    level *= 2
  return arrays


def _merge_net(arrays, greater, desc, rows, cols):
  shape = arrays[0].shape
  dm = jnp.full(shape, desc, jnp.bool_)
  dist = (rows * cols) // 2
  while dist >= 1:
    arrays = _stage_dir(arrays, greater, dist, dm, rows, cols)
    dist //= 2
  return arrays


def _score_idx_greater(selfs, parts):
  s, i = selfs[0], selfs[1]
  ps, pi = parts[0], parts[1]
  return (s > ps) | ((s == ps) & (i < pi))


def _sort_kernel(scores_ref, ts_ref, ti_ref):
  s = scores_ref[...]
  s = jnp.where(s > SCORE_THRESHOLD, s, NEG)
  idx = (lax.broadcasted_iota(jnp.int32, (SR, SC_), 0) * SC_
         + lax.broadcasted_iota(jnp.int32, (SR, SC_), 1))
  s, idx = _block_sort([s, idx], _score_idx_greater, True, SR, SC_)
  ts_ref[...] = s[:KR]
  ti_ref[...] = jnp.reshape(idx[:KR], (K,))


def _topk_sorted(scores):
  pad = jnp.full((NS - N_BOXES,), 0.0, jnp.float32)
  s2d = jnp.concatenate([scores, pad]).reshape(SR, SC_)
  return pl.pallas_call(
      _sort_kernel,
      out_shape=[
          jax.ShapeDtypeStruct((KR, KC), jnp.float32),
          jax.ShapeDtypeStruct((K,), jnp.int32),
      ],
  )(s2d)


_NG = 4 * K


@functools.cache
def _make_sc_gather():
  info = plsc.get_sparse_core_info()
  nw = info.num_cores * info.num_subcores
  b_per_w = K // nw
  mesh = plsc.VectorSubcoreMesh(core_axis_name="c", subcore_axis_name="s")

  @functools.partial(
      pl.kernel,
      mesh=mesh,
      out_type=jax.ShapeDtypeStruct((_NG,), jnp.float32),
      scratch_types=[
          pltpu.VMEM((b_per_w,), jnp.int32),
          [pltpu.VMEM((b_per_w,), jnp.float32) for _ in range(4)],
          pltpu.SemaphoreType.DMA,
      ],
  )
  def gather(x1h, y1h, x2h, y2h, idx_hbm, out_hbm, idx_v, vals, sem):
    wid = lax.axis_index("s") * info.num_cores + lax.axis_index("c")
    base = wid * b_per_w
    pltpu.sync_copy(idx_hbm.at[pl.ds(base, b_per_w)], idx_v)
    cps = [pltpu.async_copy(src.at[idx_v], buf, sem)
           for src, buf in zip((x1h, y1h, x2h, y2h), vals)]
    for cp in cps:
      cp.wait()
    for c, buf in enumerate(vals):
      pltpu.sync_copy(buf, out_hbm.at[pl.ds(c * K + base, b_per_w)])

  return gather


def _gather_topk_boxes(coords, idx_flat):
  return _make_sc_gather()(*coords, idx_flat)


def _iou_block(x1c, y1c, x2c, y2c, ac, x1r, y1r, x2r, y2r, ar):
  ix1 = jnp.maximum(x1c, x1r)
  iy1 = jnp.maximum(y1c, y1r)
  ix2 = jnp.minimum(x2c, x2r)
  iy2 = jnp.minimum(y2c, y2r)
  iw = jnp.clip(ix2 - ix1, 0.0)
  ih = jnp.clip(iy2 - iy1, 0.0)
  inter = iw * ih
  union = ac + ar - inter
  return inter / jnp.maximum(union, 1e-8)


def _sel_greater(selfs, parts):
  s, p = selfs[0], selfs[1]
  ps, pp = parts[0], parts[1]
  return (s > ps) | ((s == ps) & (p < pp))


def _nms_kernel(ts_ref, g_ref, out_ref, keep_ref):
  ts = ts_ref[...]
  x1 = jnp.reshape(g_ref[pl.ds(0 * K, K)], (KR, KC))
  y1 = jnp.reshape(g_ref[pl.ds(1 * K, K)], (KR, KC))
  x2 = jnp.reshape(g_ref[pl.ds(2 * K, K)], (KR, KC))
  y2 = jnp.reshape(g_ref[pl.ds(3 * K, K)], (KR, KC))
  area = (x2 - x1) * (y2 - y1)

  pos = (lax.broadcasted_iota(jnp.int32, (KR, KC), 0) * KC
         + lax.broadcasted_iota(jnp.int32, (KR, KC), 1))
  valid = pos < PRE_NMS_TOP_N
  lane = lax.broadcasted_iota(jnp.int32, (1, KC), 1)
  keep_ref[...] = valid.astype(jnp.int32)

  def block_body(carry):
    b, cnt = carry
    x1r = jnp.reshape(g_ref[pl.ds(0 * K + b * KC, KC)], (1, KC))
    y1r = jnp.reshape(g_ref[pl.ds(1 * K + b * KC, KC)], (1, KC))
    x2r = jnp.reshape(g_ref[pl.ds(2 * K + b * KC, KC)], (1, KC))
    y2r = jnp.reshape(g_ref[pl.ds(3 * K + b * KC, KC)], (1, KC))
    arr = (x2r - x1r) * (y2r - y1r)
    x1c, y1c, x2c, y2c, acc = (jnp.transpose(v)
                               for v in (x1r, y1r, x2r, y2r, arr))
    iou_bb = _iou_block(x1c, y1c, x2c, y2c, acc, x1r, y1r, x2r, y2r, arr)

    s_keep0 = (keep_ref[pl.ds(b, 1), :] > 0).astype(jnp.float32)

    sub_i = lax.broadcasted_iota(jnp.int32, (KC, KC), 0)
    lan_i = lax.broadcasted_iota(jnp.int32, (KC, KC), 1)
    adjm = (iou_bb > NMS_IOU_THRESHOLD) & (sub_i < lan_i)

    def jac_body(c):
      kf, _ = c
      kcol = jnp.transpose(kf)
      hit = jnp.any(adjm & (kcol > 0.5), axis=0, keepdims=True)
      knew = jnp.where(hit, 0.0, s_keep0)
      return knew, jnp.sum(jnp.abs(knew - kf)) > 0.0

    skf, _ = lax.while_loop(lambda c: c[1], jac_body, (s_keep0, True))
    s_keep = skf > 0.5

    skc = jnp.transpose(s_keep)
    for rr in range(KR):
      @pl.when(rr > b)
      def _():
        x1t, y1t, x2t, y2t, art = (a[rr:rr + 1]
                                   for a in (x1, y1, x2, y2, area))
        iou_c = _iou_block(x1c, y1c, x2c, y2c, acc, x1t, y1t, x2t, y2t, art)
        hit = jnp.any((iou_c > NMS_IOU_THRESHOLD) & skc, axis=0,
                      keepdims=True)
        krow = keep_ref[rr:rr + 1, :] > 0
        keep_ref[rr:rr + 1, :] = (krow & ~hit).astype(jnp.int32)
    keep_ref[pl.ds(b, 1), :] = s_keep.astype(jnp.int32)
    cnt = cnt + jnp.sum(skf)
    return b + 1, cnt

  def block_cond(carry):
    b, cnt = carry
    return (b < KR) & (cnt < POST_NMS_TOP_N)

  b_fin, _ = lax.while_loop(block_cond, block_body, (0, 0.0))

  keep = keep_ref[...] > 0
  sel = jnp.where(keep & (pos < b_fin * KC), ts, NEG)
  ssel, _, sx1, sy1, sx2, sy2 = _block_sort(
      [sel, pos, x1, y1, x2, y2], _sel_greater, True, KR, KC)

  zero = jnp.zeros((3, KC), jnp.float32)
  out_ref[...] = jnp.concatenate(
      [sx1[:1], sy1[:1], sx2[:1], sy2[:1], ssel[:1], zero], axis=0)


def _nms(ts, g2d):
  return pl.pallas_call(
      _nms_kernel,
      out_shape=jax.ShapeDtypeStruct((8, KC), jnp.float32),
      scratch_shapes=[
          pltpu.VMEM((KR, KC), jnp.int32),
      ],
  )(ts, g2d)


def kernel(boxes, scores):
  ts, ti = _topk_sorted(scores)
  coords = tuple(boxes[:, i] for i in range(4))
  g = _gather_topk_boxes(coords, ti)
  outb = _nms(ts, g)
  final_boxes = jnp.stack(
      [outb[0, :POST_NMS_TOP_N], outb[1, :POST_NMS_TOP_N],
       outb[2, :POST_NMS_TOP_N], outb[3, :POST_NMS_TOP_N]], axis=1)
  final_scores = outb[4, :POST_NMS_TOP_N]
  return final_boxes, final_scores


# speedup vs baseline: 1.1499x; 1.0640x over previous
import functools

import jax
import jax.numpy as jnp
from jax import lax
from jax.experimental import pallas as pl
from jax.experimental.pallas import tpu as pltpu
from jax.experimental.pallas import tpu_sc as plsc

N_BOXES = 20000
PRE_NMS_TOP_N = 2000
POST_NMS_TOP_N = 100
NMS_IOU_THRESHOLD = 0.7
SCORE_THRESHOLD = 0.1

NS = 32768
SR, SC_ = 256, 128
KR, KC = 16, 128
K = KR * KC
NEG = -jnp.inf


def _roll(x, s, axis):
  n = x.shape[axis]
  s = s % n
  if s == 0:
    return x
  if axis == 0:
    return jnp.concatenate([x[s:], x[:s]], axis=0)
  return jnp.concatenate([x[:, s:], x[:, :s]], axis=1)


def _bit_mask(shape, dist, rows, cols):
  if dist < cols:
    it = lax.broadcasted_iota(jnp.int32, shape, 1)
    return (it & dist) == 0
  it = lax.broadcasted_iota(jnp.int32, shape, 0)
  return (it & (dist // cols)) == 0


def _partner(x, dist, cols, first_mask):
  if dist < cols:
    fwd = _roll(x, dist, 1)
    bwd = _roll(x, -dist, 1)
  else:
    fwd = _roll(x, dist // cols, 0)
    bwd = _roll(x, -(dist // cols), 0)
  return jnp.where(first_mask, fwd, bwd)


def _stage_dir(arrays, greater, dist, dirmask, rows, cols):
  shape = arrays[0].shape
  first = _bit_mask(shape, dist, rows, cols)
  parts = [_partner(a, dist, cols, first) for a in arrays]
  self_gt = greater(arrays, parts)
  want_self_gt = dirmask == first
  take = want_self_gt != self_gt
  return [jnp.where(take, p, a) for a, p in zip(arrays, parts)]


def _block_sort(arrays, greater, desc, rows, cols):
  n = rows * cols
  shape = arrays[0].shape
  level = 2
  while level <= n:
    dm = _bit_mask(shape, level, rows, cols)
    if not desc:
      dm = ~dm
    dist = level // 2
    while dist >= 1:
      arrays = _stage_dir(arrays, greater, dist, dm, rows, cols)
      dist //= 2
    level *= 2
  return arrays


def _merge_net(arrays, greater, desc, rows, cols):
  shape = arrays[0].shape
  dm = jnp.full(shape, desc, jnp.bool_)
  dist = (rows * cols) // 2
  while dist >= 1:
    arrays = _stage_dir(arrays, greater, dist, dm, rows, cols)
    dist //= 2
  return arrays


def _score_idx_greater(selfs, parts):
  s, i = selfs[0], selfs[1]
  ps, pi = parts[0], parts[1]
  return (s > ps) | ((s == ps) & (i < pi))


def _sort_kernel(scores_ref, ts_ref, ti_ref):
  s = scores_ref[...]
  s = jnp.where(s > SCORE_THRESHOLD, s, NEG)
  idx = (lax.broadcasted_iota(jnp.int32, (SR, SC_), 0) * SC_
         + lax.broadcasted_iota(jnp.int32, (SR, SC_), 1))
  arrays = [s, idx]
  level = 2
  while level <= K:
    dm = _bit_mask((SR, SC_), level, SR, SC_)
    dist = level // 2
    while dist >= 1:
      arrays = _stage_dir(arrays, _score_idx_greater, dist, dm, SR, SC_)
      dist //= 2
    level *= 2
  rows = SR
  rpr = K // SC_
  while rows > KR:
    dm2 = _bit_mask((rows, SC_), 2 * K, rows, SC_)
    arrays = _stage_dir(arrays, _score_idx_greater, K, dm2, rows, SC_)
    sl = []
    for g in range(rows // (2 * rpr)):
      lo = g * 2 * rpr + (0 if g % 2 == 0 else rpr)
      sl.append((lo, lo + rpr))
    arrays = [jnp.concatenate([a[lo:hi] for lo, hi in sl], axis=0)
              for a in arrays]
    rows //= 2
    dm3 = _bit_mask((rows, SC_), K, rows, SC_)
    dist = K // 2
    while dist >= 1:
      arrays = _stage_dir(arrays, _score_idx_greater, dist, dm3, rows, SC_)
      dist //= 2
  ts_ref[...] = arrays[0]
  ti_ref[...] = jnp.reshape(arrays[1], (K,))


def _topk_sorted(scores):
  pad = jnp.full((NS - N_BOXES,), 0.0, jnp.float32)
  s2d = jnp.concatenate([scores, pad]).reshape(SR, SC_)
  return pl.pallas_call(
      _sort_kernel,
      out_shape=[
          jax.ShapeDtypeStruct((KR, KC), jnp.float32),
          jax.ShapeDtypeStruct((K,), jnp.int32),
      ],
  )(s2d)


_NG = 4 * K


@functools.cache
def _make_sc_gather():
  info = plsc.get_sparse_core_info()
  nw = info.num_cores * info.num_subcores
  b_per_w = K // nw
  mesh = plsc.VectorSubcoreMesh(core_axis_name="c", subcore_axis_name="s")

  @functools.partial(
      pl.kernel,
      mesh=mesh,
      out_type=jax.ShapeDtypeStruct((_NG,), jnp.float32),
      scratch_types=[
          pltpu.VMEM((b_per_w,), jnp.int32),
          [pltpu.VMEM((b_per_w,), jnp.float32) for _ in range(4)],
          pltpu.SemaphoreType.DMA,
      ],
  )
  def gather(x1h, y1h, x2h, y2h, idx_hbm, out_hbm, idx_v, vals, sem):
    wid = lax.axis_index("s") * info.num_cores + lax.axis_index("c")
    base = wid * b_per_w
    pltpu.sync_copy(idx_hbm.at[pl.ds(base, b_per_w)], idx_v)
    cps = [pltpu.async_copy(src.at[idx_v], buf, sem)
           for src, buf in zip((x1h, y1h, x2h, y2h), vals)]
    for cp in cps:
      cp.wait()
    for c, buf in enumerate(vals):
      pltpu.sync_copy(buf, out_hbm.at[pl.ds(c * K + base, b_per_w)])

  return gather


def _gather_topk_boxes(coords, idx_flat):
  return _make_sc_gather()(*coords, idx_flat)


def _iou_block(x1c, y1c, x2c, y2c, ac, x1r, y1r, x2r, y2r, ar):
  ix1 = jnp.maximum(x1c, x1r)
  iy1 = jnp.maximum(y1c, y1r)
  ix2 = jnp.minimum(x2c, x2r)
  iy2 = jnp.minimum(y2c, y2r)
  iw = jnp.clip(ix2 - ix1, 0.0)
  ih = jnp.clip(iy2 - iy1, 0.0)
  inter = iw * ih
  union = ac + ar - inter
  return inter / jnp.maximum(union, 1e-8)


def _sel_greater(selfs, parts):
  s, p = selfs[0], selfs[1]
  ps, pp = parts[0], parts[1]
  return (s > ps) | ((s == ps) & (p < pp))


def _nms_kernel(ts_ref, g_ref, out_ref, keep_ref):
  ts = ts_ref[...]
  x1 = jnp.reshape(g_ref[pl.ds(0 * K, K)], (KR, KC))
  y1 = jnp.reshape(g_ref[pl.ds(1 * K, K)], (KR, KC))
  x2 = jnp.reshape(g_ref[pl.ds(2 * K, K)], (KR, KC))
  y2 = jnp.reshape(g_ref[pl.ds(3 * K, K)], (KR, KC))
  area = (x2 - x1) * (y2 - y1)

  pos = (lax.broadcasted_iota(jnp.int32, (KR, KC), 0) * KC
         + lax.broadcasted_iota(jnp.int32, (KR, KC), 1))
  valid = pos < PRE_NMS_TOP_N
  lane = lax.broadcasted_iota(jnp.int32, (1, KC), 1)
  keep_ref[...] = valid.astype(jnp.int32)

  def block_body(carry):
    b, cnt = carry
    x1r = jnp.reshape(g_ref[pl.ds(0 * K + b * KC, KC)], (1, KC))
    y1r = jnp.reshape(g_ref[pl.ds(1 * K + b * KC, KC)], (1, KC))
    x2r = jnp.reshape(g_ref[pl.ds(2 * K + b * KC, KC)], (1, KC))
    y2r = jnp.reshape(g_ref[pl.ds(3 * K + b * KC, KC)], (1, KC))
    arr = (x2r - x1r) * (y2r - y1r)
    x1c, y1c, x2c, y2c, acc = (jnp.transpose(v)
                               for v in (x1r, y1r, x2r, y2r, arr))
    iou_bb = _iou_block(x1c, y1c, x2c, y2c, acc, x1r, y1r, x2r, y2r, arr)

    s_keep0 = (keep_ref[pl.ds(b, 1), :] > 0).astype(jnp.float32)

    sub_i = lax.broadcasted_iota(jnp.int32, (KC, KC), 0)
    lan_i = lax.broadcasted_iota(jnp.int32, (KC, KC), 1)
    adjm = (iou_bb > NMS_IOU_THRESHOLD) & (sub_i < lan_i)

    def jac_body(c):
      kf, _ = c
      kcol = jnp.transpose(kf)
      hit = jnp.any(adjm & (kcol > 0.5), axis=0, keepdims=True)
      knew = jnp.where(hit, 0.0, s_keep0)
      return knew, jnp.sum(jnp.abs(knew - kf)) > 0.0

    skf, _ = lax.while_loop(lambda c: c[1], jac_body, (s_keep0, True))
    s_keep = skf > 0.5

    skc = jnp.transpose(s_keep)
    for rr in range(KR):
      @pl.when(rr > b)
      def _():
        x1t, y1t, x2t, y2t, art = (a[rr:rr + 1]
                                   for a in (x1, y1, x2, y2, area))
        iou_c = _iou_block(x1c, y1c, x2c, y2c, acc, x1t, y1t, x2t, y2t, art)
        hit = jnp.any((iou_c > NMS_IOU_THRESHOLD) & skc, axis=0,
                      keepdims=True)
        krow = keep_ref[rr:rr + 1, :] > 0
        keep_ref[rr:rr + 1, :] = (krow & ~hit).astype(jnp.int32)
    keep_ref[pl.ds(b, 1), :] = s_keep.astype(jnp.int32)
    cnt = cnt + jnp.sum(skf)
    return b + 1, cnt

  def block_cond(carry):
    b, cnt = carry
    return (b < KR) & (cnt < POST_NMS_TOP_N)

  b_fin, _ = lax.while_loop(block_cond, block_body, (0, 0.0))

  keep = keep_ref[...] > 0
  sel = jnp.where(keep & (pos < b_fin * KC), ts, NEG)
  ssel, _, sx1, sy1, sx2, sy2 = _block_sort(
      [sel, pos, x1, y1, x2, y2], _sel_greater, True, KR, KC)

  zero = jnp.zeros((3, KC), jnp.float32)
  out_ref[...] = jnp.concatenate(
      [sx1[:1], sy1[:1], sx2[:1], sy2[:1], ssel[:1], zero], axis=0)


def _nms(ts, g2d):
  return pl.pallas_call(
      _nms_kernel,
      out_shape=jax.ShapeDtypeStruct((8, KC), jnp.float32),
      scratch_shapes=[
          pltpu.VMEM((KR, KC), jnp.int32),
      ],
  )(ts, g2d)


def kernel(boxes, scores):
  ts, ti = _topk_sorted(scores)
  coords = tuple(boxes[:, i] for i in range(4))
  g = _gather_topk_boxes(coords, ti)
  outb = _nms(ts, g)
  final_boxes = jnp.stack(
      [outb[0, :POST_NMS_TOP_N], outb[1, :POST_NMS_TOP_N],
       outb[2, :POST_NMS_TOP_N], outb[3, :POST_NMS_TOP_N]], axis=1)
  final_scores = outb[4, :POST_NMS_TOP_N]
  return final_boxes, final_scores
